# baseline (speedup 1.0000x reference)
import functools

import jax
import jax.numpy as jnp
from jax import lax
from jax.experimental import pallas as pl
from jax.experimental.pallas import tpu as pltpu
from jax.experimental.pallas import tpu_sc as plsc

N = 10000
E = 320000
D_IN = 128
D_H = 64
D_OUT = 32
D_T1 = 80
LANES = 128

NC = 2
NS = 16
NW = NC * NS
EPW = E // NW
BSZ = 80
NBLK = EPW // BSZ
RPT = N // NS
NB = 4

_SC_MESH = plsc.VectorSubcoreMesh(
    core_axis_name="c", subcore_axis_name="s", num_cores=NC, num_subcores=NS)


def _make_segsum(D, table_w, stage):
  ZF = RPT // BSZ
  ZR = RPT - ZF * BSZ

  def body(table, ei, out, idx_s, idx_d, rows, tbl, acc, gsem, ssem, isem):
    c = lax.axis_index("c")
    s = lax.axis_index("s")
    wid = s * NC + c
    e0 = wid * EPW
    pltpu.sync_copy(ei.at[0, pl.ds(e0, EPW)], idx_s)

    def iload(j, carry):
      pltpu.async_copy(ei.at[1, pl.ds(e0 + j * BSZ, BSZ)], idx_d.at[j], isem)
      return carry

    lax.fori_loop(0, NBLK, iload, 0)
    r0 = s * RPT
    if stage:
      pltpu.sync_copy(table.at[pl.ds(r0, RPT), pl.ds(0, D)],
                      tbl.at[pl.ds(r0, RPT)])
      gsrc = tbl
    else:
      gsrc = table

    def zfill(i, carry):
      r = i // (D // 16)
      k = lax.rem(i, D // 16) * 16
      rows[0, r, pl.ds(k, 16)] = jnp.zeros((16,), jnp.float32)
      return carry

    lax.fori_loop(0, BSZ * (D // 16), zfill, 0)
    for t in range(ZF):
      pltpu.sync_copy(rows.at[0], acc.at[pl.ds(r0 + t * BSZ, BSZ)])
    if ZR:
      pltpu.sync_copy(rows.at[0, pl.ds(0, ZR)],
                      acc.at[pl.ds(r0 + ZF * BSZ, ZR)])

    def idrain(j, carry):
      pltpu.make_async_copy(
          ei.at[1, pl.ds(e0 + j * BSZ, BSZ)], idx_d.at[j], isem).wait()
      return carry

    lax.fori_loop(0, NBLK, idrain, 0)
    plsc.subcore_barrier()

    for p in range(NB - 1):
      pltpu.async_copy(gsrc.at[idx_s.at[pl.ds(p * BSZ, BSZ)]], rows.at[p],
                       gsem.at[p])

    def step(j, carry):
      b0 = lax.rem(j, NB)
      bn = lax.rem(j + NB - 1, NB)

      @pl.when(j >= 1)
      def _():
        pltpu.make_async_copy(
            rows.at[bn], acc.at[idx_d.at[j - 1]], ssem.at[bn]).wait()

      @pl.when(j + NB - 1 < NBLK)
      def _():
        pltpu.async_copy(
            gsrc.at[idx_s.at[pl.ds((j + NB - 1) * BSZ, BSZ)]], rows.at[bn],
            gsem.at[bn])

      pltpu.make_async_copy(
          gsrc.at[idx_s.at[pl.ds(j * BSZ, BSZ)]], rows.at[b0],
          gsem.at[b0]).wait()
      pltpu.async_copy(rows.at[b0], acc.at[idx_d.at[j]], ssem.at[b0],
                       add=True)
      return carry

    lax.fori_loop(0, NBLK, step, 0)
    lbuf = (NBLK - 1) % NB
    pltpu.make_async_copy(
        rows.at[lbuf], acc.at[idx_d.at[NBLK - 1]], ssem.at[lbuf]).wait()
    plsc.subcore_barrier()
    pltpu.sync_copy(acc.at[pl.ds(r0, RPT)], out.at[c, pl.ds(r0, RPT),
                                                   pl.ds(0, D)])

  return pl.kernel(
      body,
      out_type=jax.ShapeDtypeStruct((NC, N, LANES), jnp.float32),
      mesh=_SC_MESH,
      scratch_types=[
          pltpu.VMEM((EPW,), jnp.int32),
          pltpu.VMEM((NBLK, BSZ), jnp.int32),
          pltpu.VMEM((NB, BSZ, D), jnp.float32),
          pltpu.VMEM_SHARED((N, D) if stage else (8, D),
                            jnp.float32),
          pltpu.VMEM_SHARED((N, D), jnp.float32),
          pltpu.SemaphoreType.DMA((NB,)),
          pltpu.SemaphoreType.DMA((NB,)),
          pltpu.SemaphoreType.DMA,
      ],
      compiler_params=pltpu.CompilerParams(use_tc_tiling_on_sc=False))


_segsum80 = _make_segsum(D_T1, D_T1, stage=False)
_segsum32 = _make_segsum(D_OUT, LANES, stage=True)

_BN = 10000
_GRID = N // _BN


def _proj1_body(x_ref, wl_ref, wr_ref, b_ref, aug_ref, r_ref):
  xb = x_ref[...]
  y = jnp.dot(xb, wl_ref[...], preferred_element_type=jnp.float32)
  aug_ref[...] = jnp.concatenate(
      [y,
       jnp.ones((_BN, 1), jnp.float32),
       jnp.zeros((_BN, D_T1 - D_H - 1), jnp.float32)], axis=1)
  r_ref[...] = (jnp.dot(xb, wr_ref[...], preferred_element_type=jnp.float32)
                + b_ref[...][None, :])


_proj1 = pl.pallas_call(
    _proj1_body,
    grid=(_GRID,),
    in_specs=[
        pl.BlockSpec((_BN, D_IN), lambda i: (i, 0)),
        pl.BlockSpec((D_IN, D_H), lambda i: (0, 0)),
        pl.BlockSpec((D_IN, D_H), lambda i: (0, 0)),
        pl.BlockSpec((D_H,), lambda i: (0,)),
    ],
    out_specs=[
        pl.BlockSpec((_BN, D_T1), lambda i: (i, 0)),
        pl.BlockSpec((_BN, D_H), lambda i: (i, 0)),
    ],
    out_shape=[
        jax.ShapeDtypeStruct((N, D_T1), jnp.float32),
        jax.ShapeDtypeStruct((N, D_H), jnp.float32),
    ],
)


def _mid_body(s_ref, r_ref, wl_ref, wr_ref, b_ref, y_ref, r2_ref, cnt_ref):
  ssum = s_ref[0, :, :D_H] + s_ref[1, :, :D_H]
  cnt = jnp.maximum(s_ref[0, :, D_H:D_H + 1] + s_ref[1, :, D_H:D_H + 1], 1.0)
  h = jnp.maximum(ssum / cnt + r_ref[...], 0.0)
  y2 = jnp.dot(h, wl_ref[...], preferred_element_type=jnp.float32)
  y_ref[...] = jnp.concatenate(
      [y2, jnp.zeros((_BN, LANES - D_OUT), jnp.float32)], axis=1)
  r2_ref[...] = (jnp.dot(h, wr_ref[...], preferred_element_type=jnp.float32)
                 + b_ref[...][None, :])
  cnt_ref[...] = cnt


_mid = pl.pallas_call(
    _mid_body,
    grid=(_GRID,),
    in_specs=[
        pl.BlockSpec((NC, _BN, LANES), lambda i: (0, i, 0)),
        pl.BlockSpec((_BN, D_H), lambda i: (i, 0)),
        pl.BlockSpec((D_H, D_OUT), lambda i: (0, 0)),
        pl.BlockSpec((D_H, D_OUT), lambda i: (0, 0)),
        pl.BlockSpec((D_OUT,), lambda i: (0,)),
    ],
    out_specs=[
        pl.BlockSpec((_BN, LANES), lambda i: (i, 0)),
        pl.BlockSpec((_BN, D_OUT), lambda i: (i, 0)),
        pl.BlockSpec((_BN, 1), lambda i: (i, 0)),
    ],
    out_shape=[
        jax.ShapeDtypeStruct((N, LANES), jnp.float32),
        jax.ShapeDtypeStruct((N, D_OUT), jnp.float32),
        jax.ShapeDtypeStruct((N, 1), jnp.float32),
    ],
)


def _fin_body(s_ref, c_ref, r_ref, o_ref):
  ssum = s_ref[0, :, :D_OUT] + s_ref[1, :, :D_OUT]
  o_ref[...] = ssum / c_ref[...] + r_ref[...]


_fin = pl.pallas_call(
    _fin_body,
    grid=(_GRID,),
    in_specs=[
        pl.BlockSpec((NC, _BN, LANES), lambda i: (0, i, 0)),
        pl.BlockSpec((_BN, 1), lambda i: (i, 0)),
        pl.BlockSpec((_BN, D_OUT), lambda i: (i, 0)),
    ],
    out_specs=pl.BlockSpec((_BN, D_OUT), lambda i: (i, 0)),
    out_shape=jax.ShapeDtypeStruct((N, D_OUT), jnp.float32),
)


@jax.jit
def _run(x, edge_index, W1_l, W1_r, b1, W2_l, W2_r, b2):
  y1aug, r1 = _proj1(x, W1_l, W1_r, b1)
  sum1 = _segsum80(y1aug, edge_index)
  y2, r2, cnt = _mid(sum1, r1, W2_l, W2_r, b2)
  sum2 = _segsum32(y2, edge_index)
  return _fin(sum2, cnt, r2)


def kernel(x, edge_index, W1_l, W1_r, b1, W2_l, W2_r, b2):
  return _run(x, edge_index, W1_l, W1_r, b1, W2_l, W2_r, b2)


# speedup vs baseline: 1.0187x; 1.0187x over previous
import functools

import jax
import jax.numpy as jnp
from jax import lax
from jax.experimental import pallas as pl
from jax.experimental.pallas import tpu as pltpu
from jax.experimental.pallas import tpu_sc as plsc

N = 10000
E = 320000
D_IN = 128
D_H = 64
D_OUT = 32
D_T1 = 80
LANES = 128

NC = 2
NS = 16
NW = NC * NS
EPW = E // NW
BSZ = 80
NBLK = EPW // BSZ
RPT = N // NS
NB = 5

_SC_MESH = plsc.VectorSubcoreMesh(
    core_axis_name="c", subcore_axis_name="s", num_cores=NC, num_subcores=NS)


def _make_segsum(D, table_w, stage):
  ZF = RPT // BSZ
  ZR = RPT - ZF * BSZ

  def body(table, ei, out, idx_s, idx_d, rows, tbl, acc, gsem, ssem, isem):
    c = lax.axis_index("c")
    s = lax.axis_index("s")
    wid = s * NC + c
    e0 = wid * EPW
    pltpu.sync_copy(ei.at[0, pl.ds(e0, EPW)], idx_s)

    def iload(j, carry):
      pltpu.async_copy(ei.at[1, pl.ds(e0 + j * BSZ, BSZ)], idx_d.at[j], isem)
      return carry

    lax.fori_loop(0, NBLK, iload, 0)
    r0 = s * RPT
    if stage:
      pltpu.sync_copy(table.at[pl.ds(r0, RPT), pl.ds(0, D)],
                      tbl.at[pl.ds(r0, RPT)])
      gsrc = tbl
    else:
      gsrc = table

    def zfill(i, carry):
      r = i // (D // 16)
      k = lax.rem(i, D // 16) * 16
      rows[0, r, pl.ds(k, 16)] = jnp.zeros((16,), jnp.float32)
      return carry

    lax.fori_loop(0, BSZ * (D // 16), zfill, 0)
    for t in range(ZF):
      pltpu.sync_copy(rows.at[0], acc.at[pl.ds(r0 + t * BSZ, BSZ)])
    if ZR:
      pltpu.sync_copy(rows.at[0, pl.ds(0, ZR)],
                      acc.at[pl.ds(r0 + ZF * BSZ, ZR)])

    def idrain(j, carry):
      pltpu.make_async_copy(
          ei.at[1, pl.ds(e0 + j * BSZ, BSZ)], idx_d.at[j], isem).wait()
      return carry

    lax.fori_loop(0, NBLK, idrain, 0)
    plsc.subcore_barrier()

    for p in range(NB - 1):
      pltpu.async_copy(gsrc.at[idx_s.at[pl.ds(p * BSZ, BSZ)]], rows.at[p],
                       gsem.at[p])

    def step(j, carry):
      b0 = lax.rem(j, NB)
      bn = lax.rem(j + NB - 1, NB)

      @pl.when(j >= 1)
      def _():
        pltpu.make_async_copy(
            rows.at[bn], acc.at[idx_d.at[j - 1]], ssem.at[bn]).wait()

      @pl.when(j + NB - 1 < NBLK)
      def _():
        pltpu.async_copy(
            gsrc.at[idx_s.at[pl.ds((j + NB - 1) * BSZ, BSZ)]], rows.at[bn],
            gsem.at[bn])

      pltpu.make_async_copy(
          gsrc.at[idx_s.at[pl.ds(j * BSZ, BSZ)]], rows.at[b0],
          gsem.at[b0]).wait()
      pltpu.async_copy(rows.at[b0], acc.at[idx_d.at[j]], ssem.at[b0],
                       add=True)
      return carry

    lax.fori_loop(0, NBLK, step, 0)
    lbuf = (NBLK - 1) % NB
    pltpu.make_async_copy(
        rows.at[lbuf], acc.at[idx_d.at[NBLK - 1]], ssem.at[lbuf]).wait()
    plsc.subcore_barrier()
    pltpu.sync_copy(acc.at[pl.ds(r0, RPT)], out.at[c, pl.ds(r0, RPT),
                                                   pl.ds(0, D)])

  return pl.kernel(
      body,
      out_type=jax.ShapeDtypeStruct((NC, N, LANES), jnp.float32),
      mesh=_SC_MESH,
      scratch_types=[
          pltpu.VMEM((EPW,), jnp.int32),
          pltpu.VMEM((NBLK, BSZ), jnp.int32),
          pltpu.VMEM((NB, BSZ, D), jnp.float32),
          pltpu.VMEM_SHARED((N, D) if stage else (8, D),
                            jnp.float32),
          pltpu.VMEM_SHARED((N, D), jnp.float32),
          pltpu.SemaphoreType.DMA((NB,)),
          pltpu.SemaphoreType.DMA((NB,)),
          pltpu.SemaphoreType.DMA,
      ],
      compiler_params=pltpu.CompilerParams(use_tc_tiling_on_sc=False))


_segsum80 = _make_segsum(D_T1, D_T1, stage=False)
_segsum32 = _make_segsum(D_OUT, LANES, stage=True)

_BN = 5000
_GRID = N // _BN


def _proj1_body(x_ref, wl_ref, wr_ref, b_ref, aug_ref, r_ref):
  xb = x_ref[...]
  y = jnp.dot(xb, wl_ref[...], preferred_element_type=jnp.float32)
  aug_ref[...] = jnp.concatenate(
      [y,
       jnp.ones((_BN, 1), jnp.float32),
       jnp.zeros((_BN, D_T1 - D_H - 1), jnp.float32)], axis=1)
  r_ref[...] = (jnp.dot(xb, wr_ref[...], preferred_element_type=jnp.float32)
                + b_ref[...][None, :])


_proj1 = pl.pallas_call(
    _proj1_body,
    grid=(_GRID,),
    in_specs=[
        pl.BlockSpec((_BN, D_IN), lambda i: (i, 0)),
        pl.BlockSpec((D_IN, D_H), lambda i: (0, 0)),
        pl.BlockSpec((D_IN, D_H), lambda i: (0, 0)),
        pl.BlockSpec((D_H,), lambda i: (0,)),
    ],
    out_specs=[
        pl.BlockSpec((_BN, D_T1), lambda i: (i, 0)),
        pl.BlockSpec((_BN, D_H), lambda i: (i, 0)),
    ],
    out_shape=[
        jax.ShapeDtypeStruct((N, D_T1), jnp.float32),
        jax.ShapeDtypeStruct((N, D_H), jnp.float32),
    ],
)


def _mid_body(s_ref, r_ref, wl_ref, wr_ref, b_ref, y_ref, r2_ref, cnt_ref):
  ssum = s_ref[0, :, :D_H] + s_ref[1, :, :D_H]
  cnt = jnp.maximum(s_ref[0, :, D_H:D_H + 1] + s_ref[1, :, D_H:D_H + 1], 1.0)
  h = jnp.maximum(ssum / cnt + r_ref[...], 0.0)
  y2 = jnp.dot(h, wl_ref[...], preferred_element_type=jnp.float32)
  y_ref[...] = jnp.concatenate(
      [y2, jnp.zeros((_BN, LANES - D_OUT), jnp.float32)], axis=1)
  r2_ref[...] = (jnp.dot(h, wr_ref[...], preferred_element_type=jnp.float32)
                 + b_ref[...][None, :])
  cnt_ref[...] = cnt


_mid = pl.pallas_call(
    _mid_body,
    grid=(_GRID,),
    in_specs=[
        pl.BlockSpec((NC, _BN, LANES), lambda i: (0, i, 0)),
        pl.BlockSpec((_BN, D_H), lambda i: (i, 0)),
        pl.BlockSpec((D_H, D_OUT), lambda i: (0, 0)),
        pl.BlockSpec((D_H, D_OUT), lambda i: (0, 0)),
        pl.BlockSpec((D_OUT,), lambda i: (0,)),
    ],
    out_specs=[
        pl.BlockSpec((_BN, LANES), lambda i: (i, 0)),
        pl.BlockSpec((_BN, D_OUT), lambda i: (i, 0)),
        pl.BlockSpec((_BN, 1), lambda i: (i, 0)),
    ],
    out_shape=[
        jax.ShapeDtypeStruct((N, LANES), jnp.float32),
        jax.ShapeDtypeStruct((N, D_OUT), jnp.float32),
        jax.ShapeDtypeStruct((N, 1), jnp.float32),
    ],
)


def _fin_body(s_ref, c_ref, r_ref, o_ref):
  ssum = s_ref[0, :, :D_OUT] + s_ref[1, :, :D_OUT]
  o_ref[...] = ssum / c_ref[...] + r_ref[...]


_fin = pl.pallas_call(
    _fin_body,
    grid=(_GRID,),
    in_specs=[
        pl.BlockSpec((NC, _BN, LANES), lambda i: (0, i, 0)),
        pl.BlockSpec((_BN, 1), lambda i: (i, 0)),
        pl.BlockSpec((_BN, D_OUT), lambda i: (i, 0)),
    ],
    out_specs=pl.BlockSpec((_BN, D_OUT), lambda i: (i, 0)),
    out_shape=jax.ShapeDtypeStruct((N, D_OUT), jnp.float32),
)


@jax.jit
def _run(x, edge_index, W1_l, W1_r, b1, W2_l, W2_r, b2):
  y1aug, r1 = _proj1(x, W1_l, W1_r, b1)
  sum1 = _segsum80(y1aug, edge_index)
  y2, r2, cnt = _mid(sum1, r1, W2_l, W2_r, b2)
  sum2 = _segsum32(y2, edge_index)
  return _fin(sum2, cnt, r2)


def kernel(x, edge_index, W1_l, W1_r, b1, W2_l, W2_r, b2):
  return _run(x, edge_index, W1_l, W1_r, b1, W2_l, W2_r, b2)
